# baseline (speedup 1.0000x reference)
import functools

import jax
import jax.numpy as jnp
from jax import lax
from jax.experimental import pallas as pl
from jax.experimental.pallas import tpu as pltpu
from jax.experimental.pallas import tpu_sc as plsc

M_ITEM = 1000000
DIM = 64
B = 1024
HIST = 200
N_NEG = 100

L = 16
NGRP = 7
NPAD = NGRP * L
ROWS = HIST + NPAD + 8
TGT_ROW = HIST + NPAD

NC = 2
NS = 16
NW = NC * NS
BPW = B // NW


def _shuffle(v, idx16):
    return lax.gather(
        v, idx16[:, None],
        lax.GatherDimensionNumbers(offset_dims=(), collapsed_slice_dims=(0,),
                                   start_index_map=(0,)),
        (1,), mode=lax.GatherScatterMode.PROMISE_IN_BOUNDS)


def _lanesum(v, lanes):
    for k in (8, 4, 2, 1):
        v = v + _shuffle(v, lanes ^ k)
    return v


def _merge_tree(vs, lanes):
    for k in (1, 2, 4, 8):
        nxt = []
        for i in range(0, len(vs), 2):
            a, b = vs[i], vs[i + 1]
            asum = a + _shuffle(a, lanes ^ k)
            bsum = b + _shuffle(b, lanes ^ k)
            nxt.append(jnp.where((lanes & k) == 0, asum, bsum))
        vs = nxt
    return vs[0]


def _sc_body(table, idx, scal_out, nd_out, nn_out,
             idx_v, rows_a, rows_b, scal_v, nd_v, nn_v, sem_a, sem_b):
    wid = lax.axis_index("s") * NC + lax.axis_index("c")
    base = wid * BPW

    pltpu.sync_copy(idx.at[pl.ds(base, BPW)], idx_v)

    lanes = jnp.arange(L, dtype=jnp.int32)
    zero = jnp.zeros((L,), jnp.float32)

    def fire(bi, rows_v, sem):
        for g in range(ROWS // L):
            iv = idx_v[bi, pl.ds(g * L, L)]
            pltpu.async_copy(table.at[iv], rows_v.at[pl.ds(g * L, L)], sem)

    def drain(bi, rows_v, sem):
        for g in range(ROWS // L):
            iv = idx_v[bi, pl.ds(g * L, L)]
            pltpu.make_async_copy(table.at[iv], rows_v.at[pl.ds(g * L, L)],
                                  sem).wait()

    def compute(bi, rows_v):
        t0 = rows_v[TGT_ROW, pl.ds(0, L)]
        t1 = rows_v[TGT_ROW, pl.ds(L, L)]
        t2 = rows_v[TGT_ROW, pl.ds(2 * L, L)]
        t3 = rows_v[TGT_ROW, pl.ds(3 * L, L)]

        def hist_body(h, acc):
            a0, a1, a2, a3 = acc
            for u in range(4):
                r = h * 4 + u
                a0 = a0 + rows_v[r, pl.ds(0, L)]
                a1 = a1 + rows_v[r, pl.ds(L, L)]
                a2 = a2 + rows_v[r, pl.ds(2 * L, L)]
                a3 = a3 + rows_v[r, pl.ds(3 * L, L)]
            return a0, a1, a2, a3

        a0, a1, a2, a3 = lax.fori_loop(0, HIST // 4, hist_body,
                                       (zero, zero, zero, zero))

        pos = _lanesum(a0 * t0 + a1 * t1 + a2 * t2 + a3 * t3, lanes)
        m2 = _lanesum(a0 * a0 + a1 * a1 + a2 * a2 + a3 * a3, lanes)
        tt = _lanesum(t0 * t0 + t1 * t1 + t2 * t2 + t3 * t3, lanes)
        sv = jnp.where(lanes == 0, pos,
                       jnp.where(lanes == 1, m2,
                                 jnp.where(lanes == 2, tt, zero)))
        scal_v[bi, pl.ds(0, L)] = sv

        for g in range(NGRP):
            dps = []
            nps = []
            for j in range(L):
                r = HIST + g * L + j
                n0 = rows_v[r, pl.ds(0, L)]
                n1 = rows_v[r, pl.ds(L, L)]
                n2 = rows_v[r, pl.ds(2 * L, L)]
                n3 = rows_v[r, pl.ds(3 * L, L)]
                dps.append(n0 * t0 + n1 * t1 + n2 * t2 + n3 * t3)
                nps.append(n0 * n0 + n1 * n1 + n2 * n2 + n3 * n3)
            nd_v[bi, pl.ds(g * L, L)] = _merge_tree(dps, lanes)
            nn_v[bi, pl.ds(g * L, L)] = _merge_tree(nps, lanes)

    fire(0, rows_a, sem_a)

    def pair_body(p, carry):
        b0 = 2 * p
        fire(b0 + 1, rows_b, sem_b)
        drain(b0, rows_a, sem_a)
        compute(b0, rows_a)
        fire(lax.rem(b0 + 2, BPW), rows_a, sem_a)
        drain(b0 + 1, rows_b, sem_b)
        compute(b0 + 1, rows_b)
        return carry

    lax.fori_loop(0, BPW // 2, pair_body, 0)
    drain(0, rows_a, sem_a)

    pltpu.sync_copy(scal_v, scal_out.at[pl.ds(base, BPW)])
    pltpu.sync_copy(nd_v, nd_out.at[pl.ds(base, BPW)])
    pltpu.sync_copy(nn_v, nn_out.at[pl.ds(base, BPW)])


def _sc_gather_reduce(table, idx_flat):
    mesh = plsc.VectorSubcoreMesh(core_axis_name="c", subcore_axis_name="s")
    f = pl.kernel(
        _sc_body,
        mesh=mesh,
        out_type=(
            jax.ShapeDtypeStruct((B, L), jnp.float32),
            jax.ShapeDtypeStruct((B, NPAD), jnp.float32),
            jax.ShapeDtypeStruct((B, NPAD), jnp.float32),
        ),
        scratch_types=[
            pltpu.VMEM((BPW, ROWS), jnp.int32),
            pltpu.VMEM((ROWS, DIM), jnp.float32),
            pltpu.VMEM((ROWS, DIM), jnp.float32),
            pltpu.VMEM((BPW, L), jnp.float32),
            pltpu.VMEM((BPW, NPAD), jnp.float32),
            pltpu.VMEM((BPW, NPAD), jnp.float32),
            pltpu.SemaphoreType.DMA,
            pltpu.SemaphoreType.DMA,
        ],
        compiler_params=pltpu.CompilerParams(use_tc_tiling_on_sc=False),
    )
    return f(table, idx_flat)


def _tc_body(scal_ref, nd_ref, nn_ref, out_ref):
    eps = 1e-8
    pos_dot = scal_ref[:, 0:1]
    m2 = scal_ref[:, 1:2]
    tt = scal_ref[:, 2:3]
    na = jnp.maximum(jnp.sqrt(tt), eps)
    nb = jnp.maximum(jnp.sqrt(m2) * (1.0 / HIST), eps)
    pos_sim = (pos_dot * (1.0 / HIST)) / (na * nb)

    nd = nd_ref[:]
    nn = nn_ref[:]
    nbn = jnp.maximum(jnp.sqrt(nn), eps)
    neg_sim = nd / (na * nbn)
    col = lax.broadcasted_iota(jnp.int32, (B, NPAD), 1)
    neg_sim = jnp.where(col < N_NEG, neg_sim, -1e30)

    logits = jnp.concatenate([pos_sim, neg_sim], axis=1)
    mx = jnp.max(logits, axis=1, keepdims=True)
    lse = mx + jnp.log(jnp.sum(jnp.exp(logits - mx), axis=1, keepdims=True))
    logp0 = pos_sim - lse
    out_ref[...] = jnp.reshape(-jnp.sum(logp0) * (1.0 / B), (1, 1))


def _tc_loss(scal, nd, nn):
    return pl.pallas_call(
        _tc_body,
        out_shape=jax.ShapeDtypeStruct((1, 1), jnp.float32),
    )(scal, nd, nn)


def _tc_idx_body(tr_ref, ng_ref, tg_ref, out_ref):
    z12 = jnp.zeros((B, NPAD - N_NEG), jnp.int32)
    z7 = jnp.zeros((B, ROWS - TGT_ROW - 1), jnp.int32)
    out_ref[...] = jnp.concatenate(
        [tr_ref[...], ng_ref[...], z12, tg_ref[...], z7], axis=1)


def _tc_build_idx(train_idx, neg_idx, target_idx):
    return pl.pallas_call(
        _tc_idx_body,
        out_shape=jax.ShapeDtypeStruct((B, ROWS), jnp.int32),
    )(train_idx, neg_idx, target_idx)


def kernel(table, train_idx, neg_idx, target_idx):
    idx2d = _tc_build_idx(train_idx.astype(jnp.int32),
                          neg_idx.astype(jnp.int32),
                          target_idx.astype(jnp.int32)[:, None])
    scal, nd, nn = _sc_gather_reduce(table, idx2d)
    loss = _tc_loss(scal, nd, nn)
    return jnp.reshape(loss, ())


# speedup vs baseline: 1.4513x; 1.4513x over previous
import functools

import jax
import jax.numpy as jnp
from jax import lax
from jax.experimental import pallas as pl
from jax.experimental.pallas import tpu as pltpu
from jax.experimental.pallas import tpu_sc as plsc

M_ITEM = 1000000
DIM = 64
B = 1024
HIST = 200
N_NEG = 100

L = 16
NPAD = 104
ROWS = 304
TGT_ROW = HIST + N_NEG
NEG_STARTS = (0, 16, 32, 48, 64, 80, 88)

NC = 2
NS = 16
NW = NC * NS
BPW = B // NW


def _shuffle(v, idx16):
    return lax.gather(
        v, idx16[:, None],
        lax.GatherDimensionNumbers(offset_dims=(), collapsed_slice_dims=(0,),
                                   start_index_map=(0,)),
        (1,), mode=lax.GatherScatterMode.PROMISE_IN_BOUNDS)


def _lanesum(v, lanes):
    for k in (8, 4, 2, 1):
        v = v + _shuffle(v, lanes ^ k)
    return v


def _merge_tree(vs, lanes):
    for k in (1, 2, 4, 8):
        nxt = []
        for i in range(0, len(vs), 2):
            a, b = vs[i], vs[i + 1]
            asum = a + _shuffle(a, lanes ^ k)
            bsum = b + _shuffle(b, lanes ^ k)
            nxt.append(jnp.where((lanes & k) == 0, asum, bsum))
        vs = nxt
    return vs[0]


def _sc_body(table, idx, scal_out, nd_out, nn_out,
             idx_v, rows_a, rows_b, scal_v, nd_v, nn_v, sem_a, sem_b):
    wid = lax.axis_index("s") * NC + lax.axis_index("c")
    base = wid * BPW

    pltpu.sync_copy(idx.at[pl.ds(base, BPW)], idx_v)

    lanes = jnp.arange(L, dtype=jnp.int32)
    zero = jnp.zeros((L,), jnp.float32)

    def _copies(bi, rows_v, sem):
        return (
            (idx_v.at[bi, pl.ds(0, 128)], rows_v.at[pl.ds(0, 128)], sem),
            (idx_v.at[bi, pl.ds(128, 128)], rows_v.at[pl.ds(128, 128)], sem),
            (idx_v.at[bi, pl.ds(256, 48)], rows_v.at[pl.ds(256, 48)], sem),
        )

    def fire(bi, rows_v, sem):
        for i_ref, d_ref, s in _copies(bi, rows_v, sem):
            pltpu.async_copy(table.at[i_ref], d_ref, s)

    def drain(bi, rows_v, sem):
        for i_ref, d_ref, s in _copies(bi, rows_v, sem):
            pltpu.make_async_copy(table.at[i_ref], d_ref, s).wait()

    def compute(bi, rows_v):
        t0 = rows_v[TGT_ROW, pl.ds(0, L)]
        t1 = rows_v[TGT_ROW, pl.ds(L, L)]
        t2 = rows_v[TGT_ROW, pl.ds(2 * L, L)]
        t3 = rows_v[TGT_ROW, pl.ds(3 * L, L)]

        def hist_body(h, acc):
            a0, a1, a2, a3 = acc
            for u in range(4):
                r = h * 4 + u
                a0 = a0 + rows_v[r, pl.ds(0, L)]
                a1 = a1 + rows_v[r, pl.ds(L, L)]
                a2 = a2 + rows_v[r, pl.ds(2 * L, L)]
                a3 = a3 + rows_v[r, pl.ds(3 * L, L)]
            return a0, a1, a2, a3

        a0, a1, a2, a3 = lax.fori_loop(0, HIST // 4, hist_body,
                                       (zero, zero, zero, zero))

        pos = _lanesum(a0 * t0 + a1 * t1 + a2 * t2 + a3 * t3, lanes)
        m2 = _lanesum(a0 * a0 + a1 * a1 + a2 * a2 + a3 * a3, lanes)
        tt = _lanesum(t0 * t0 + t1 * t1 + t2 * t2 + t3 * t3, lanes)
        sv = jnp.where(lanes == 0, pos,
                       jnp.where(lanes == 1, m2,
                                 jnp.where(lanes == 2, tt, zero)))
        scal_v[bi, pl.ds(0, L)] = sv

        for g0 in NEG_STARTS:
            dps = []
            nps = []
            for j in range(L):
                r = HIST + g0 + j
                n0 = rows_v[r, pl.ds(0, L)]
                n1 = rows_v[r, pl.ds(L, L)]
                n2 = rows_v[r, pl.ds(2 * L, L)]
                n3 = rows_v[r, pl.ds(3 * L, L)]
                dps.append(n0 * t0 + n1 * t1 + n2 * t2 + n3 * t3)
                nps.append(n0 * n0 + n1 * n1 + n2 * n2 + n3 * n3)
            nd_v[bi, pl.ds(g0, L)] = _merge_tree(dps, lanes)
            nn_v[bi, pl.ds(g0, L)] = _merge_tree(nps, lanes)

    fire(0, rows_a, sem_a)

    def pair_body(p, carry):
        b0 = 2 * p
        fire(b0 + 1, rows_b, sem_b)
        drain(b0, rows_a, sem_a)
        compute(b0, rows_a)
        fire(lax.rem(b0 + 2, BPW), rows_a, sem_a)
        drain(b0 + 1, rows_b, sem_b)
        compute(b0 + 1, rows_b)
        return carry

    lax.fori_loop(0, BPW // 2, pair_body, 0)
    drain(0, rows_a, sem_a)

    pltpu.sync_copy(scal_v, scal_out.at[pl.ds(base, BPW)])
    pltpu.sync_copy(nd_v, nd_out.at[pl.ds(base, BPW)])
    pltpu.sync_copy(nn_v, nn_out.at[pl.ds(base, BPW)])


def _sc_gather_reduce(table, idx_flat):
    mesh = plsc.VectorSubcoreMesh(core_axis_name="c", subcore_axis_name="s")
    f = pl.kernel(
        _sc_body,
        mesh=mesh,
        out_type=(
            jax.ShapeDtypeStruct((B, L), jnp.float32),
            jax.ShapeDtypeStruct((B, NPAD), jnp.float32),
            jax.ShapeDtypeStruct((B, NPAD), jnp.float32),
        ),
        scratch_types=[
            pltpu.VMEM((BPW, ROWS), jnp.int32),
            pltpu.VMEM((ROWS, DIM), jnp.float32),
            pltpu.VMEM((ROWS, DIM), jnp.float32),
            pltpu.VMEM((BPW, L), jnp.float32),
            pltpu.VMEM((BPW, NPAD), jnp.float32),
            pltpu.VMEM((BPW, NPAD), jnp.float32),
            pltpu.SemaphoreType.DMA,
            pltpu.SemaphoreType.DMA,
        ],
        compiler_params=pltpu.CompilerParams(use_tc_tiling_on_sc=False),
    )
    return f(table, idx_flat)


def _tc_body(scal_ref, nd_ref, nn_ref, out_ref):
    eps = 1e-8
    pos_dot = scal_ref[:, 0:1]
    m2 = scal_ref[:, 1:2]
    tt = scal_ref[:, 2:3]
    na = jnp.maximum(jnp.sqrt(tt), eps)
    nb = jnp.maximum(jnp.sqrt(m2) * (1.0 / HIST), eps)
    pos_sim = (pos_dot * (1.0 / HIST)) / (na * nb)

    nd = nd_ref[:]
    nn = nn_ref[:]
    nbn = jnp.maximum(jnp.sqrt(nn), eps)
    neg_sim = nd / (na * nbn)
    col = lax.broadcasted_iota(jnp.int32, (B, NPAD), 1)
    neg_sim = jnp.where(col < N_NEG, neg_sim, -1e30)

    logits = jnp.concatenate([pos_sim, neg_sim], axis=1)
    mx = jnp.max(logits, axis=1, keepdims=True)
    lse = mx + jnp.log(jnp.sum(jnp.exp(logits - mx), axis=1, keepdims=True))
    logp0 = pos_sim - lse
    out_ref[...] = jnp.reshape(-jnp.sum(logp0) * (1.0 / B), (1, 1))


def _tc_loss(scal, nd, nn):
    return pl.pallas_call(
        _tc_body,
        out_shape=jax.ShapeDtypeStruct((1, 1), jnp.float32),
    )(scal, nd, nn)


def _tc_idx_body(tr_ref, ng_ref, tg_ref, out_ref):
    ztail = jnp.zeros((B, ROWS - TGT_ROW - 1), jnp.int32)
    out_ref[...] = jnp.concatenate(
        [tr_ref[...], ng_ref[...], tg_ref[...], ztail], axis=1)


def _tc_build_idx(train_idx, neg_idx, target_idx):
    return pl.pallas_call(
        _tc_idx_body,
        out_shape=jax.ShapeDtypeStruct((B, ROWS), jnp.int32),
    )(train_idx, neg_idx, target_idx)


def kernel(table, train_idx, neg_idx, target_idx):
    idx2d = _tc_build_idx(train_idx.astype(jnp.int32),
                          neg_idx.astype(jnp.int32),
                          target_idx.astype(jnp.int32)[:, None])
    scal, nd, nn = _sc_gather_reduce(table, idx2d)
    loss = _tc_loss(scal, nd, nn)
    return jnp.reshape(loss, ())


# speedup vs baseline: 1.5100x; 1.0405x over previous
import functools

import jax
import jax.numpy as jnp
from jax import lax
from jax.experimental import pallas as pl
from jax.experimental.pallas import tpu as pltpu
from jax.experimental.pallas import tpu_sc as plsc

M_ITEM = 1000000
DIM = 64
B = 1024
HIST = 200
N_NEG = 100

L = 16
NPAD = 104
ROWS = 304
TGT_ROW = HIST + N_NEG
NEG_STARTS = (0, 16, 32, 48, 64, 80, 88)

NC = 2
NS = 16
NW = NC * NS
BPW = B // NW


def _shuffle(v, idx16):
    return lax.gather(
        v, idx16[:, None],
        lax.GatherDimensionNumbers(offset_dims=(), collapsed_slice_dims=(0,),
                                   start_index_map=(0,)),
        (1,), mode=lax.GatherScatterMode.PROMISE_IN_BOUNDS)


def _lanesum(v, lanes):
    for k in (8, 4, 2, 1):
        v = v + _shuffle(v, lanes ^ k)
    return v


def _merge_tree(vs, lanes):
    for k in (1, 2, 4, 8):
        nxt = []
        for i in range(0, len(vs), 2):
            a, b = vs[i], vs[i + 1]
            asum = a + _shuffle(a, lanes ^ k)
            bsum = b + _shuffle(b, lanes ^ k)
            nxt.append(jnp.where((lanes & k) == 0, asum, bsum))
        vs = nxt
    return vs[0]


def _sc_body(table, idx, scal_out, nd_out, nn_out,
             idx_v, rows_a, rows_b, scal_v, nd_v, nn_v, sem_a, sem_b):
    wid = lax.axis_index("s") * NC + lax.axis_index("c")
    base = wid * BPW

    pltpu.sync_copy(idx.at[pl.ds(base, BPW)], idx_v)

    lanes = jnp.arange(L, dtype=jnp.int32)
    zero = jnp.zeros((L,), jnp.float32)

    def _copies(bi, rows_v, sem):
        return (
            (idx_v.at[bi, pl.ds(0, 128)], rows_v.at[pl.ds(0, 128)], sem),
            (idx_v.at[bi, pl.ds(128, 128)], rows_v.at[pl.ds(128, 128)], sem),
            (idx_v.at[bi, pl.ds(256, 48)], rows_v.at[pl.ds(256, 48)], sem),
        )

    def fire(bi, rows_v, sem):
        for i_ref, d_ref, s in _copies(bi, rows_v, sem):
            pltpu.async_copy(table.at[i_ref], d_ref, s)

    def drain(bi, rows_v, sem):
        for i_ref, d_ref, s in _copies(bi, rows_v, sem):
            pltpu.make_async_copy(table.at[i_ref], d_ref, s).wait()

    def compute(bi, rows_v):
        t0 = rows_v[TGT_ROW, pl.ds(0, L)]
        t1 = rows_v[TGT_ROW, pl.ds(L, L)]
        t2 = rows_v[TGT_ROW, pl.ds(2 * L, L)]
        t3 = rows_v[TGT_ROW, pl.ds(3 * L, L)]

        def hist_body(h, acc):
            a0, a1, a2, a3 = acc
            for u in range(4):
                r = h * 4 + u
                a0 = a0 + rows_v[r, pl.ds(0, L)]
                a1 = a1 + rows_v[r, pl.ds(L, L)]
                a2 = a2 + rows_v[r, pl.ds(2 * L, L)]
                a3 = a3 + rows_v[r, pl.ds(3 * L, L)]
            return a0, a1, a2, a3

        a0, a1, a2, a3 = lax.fori_loop(0, HIST // 4, hist_body,
                                       (zero, zero, zero, zero))

        pos = _lanesum(a0 * t0 + a1 * t1 + a2 * t2 + a3 * t3, lanes)
        m2 = _lanesum(a0 * a0 + a1 * a1 + a2 * a2 + a3 * a3, lanes)
        tt = _lanesum(t0 * t0 + t1 * t1 + t2 * t2 + t3 * t3, lanes)
        sv = jnp.where(lanes == 0, pos,
                       jnp.where(lanes == 1, m2,
                                 jnp.where(lanes == 2, tt, zero)))
        scal_v[bi, pl.ds(0, L)] = sv

        for g0 in NEG_STARTS:
            dps = []
            nps = []
            for j in range(L):
                r = HIST + g0 + j
                n0 = rows_v[r, pl.ds(0, L)]
                n1 = rows_v[r, pl.ds(L, L)]
                n2 = rows_v[r, pl.ds(2 * L, L)]
                n3 = rows_v[r, pl.ds(3 * L, L)]
                dps.append(n0 * t0 + n1 * t1 + n2 * t2 + n3 * t3)
                nps.append(n0 * n0 + n1 * n1 + n2 * n2 + n3 * n3)
            nd_v[bi, pl.ds(g0, L)] = _merge_tree(dps, lanes)
            nn_v[bi, pl.ds(g0, L)] = _merge_tree(nps, lanes)

    fire(0, rows_a, sem_a)

    def pair_body(p, carry):
        b0 = 2 * p
        fire(b0 + 1, rows_b, sem_b)
        drain(b0, rows_a, sem_a)
        compute(b0, rows_a)
        fire(lax.rem(b0 + 2, BPW), rows_a, sem_a)
        drain(b0 + 1, rows_b, sem_b)
        compute(b0 + 1, rows_b)
        return carry

    lax.fori_loop(0, BPW // 2, pair_body, 0)
    drain(0, rows_a, sem_a)

    pltpu.sync_copy(scal_v, scal_out.at[pl.ds(base, BPW)])
    pltpu.sync_copy(nd_v, nd_out.at[pl.ds(base, BPW)])
    pltpu.sync_copy(nn_v, nn_out.at[pl.ds(base, BPW)])


def _sc_gather_reduce(table, idx_flat):
    mesh = plsc.VectorSubcoreMesh(core_axis_name="c", subcore_axis_name="s")
    f = pl.kernel(
        _sc_body,
        mesh=mesh,
        out_type=(
            jax.ShapeDtypeStruct((B, L), jnp.float32),
            jax.ShapeDtypeStruct((B, NPAD), jnp.float32),
            jax.ShapeDtypeStruct((B, NPAD), jnp.float32),
        ),
        scratch_types=[
            pltpu.VMEM((BPW, ROWS), jnp.int32),
            pltpu.VMEM((ROWS, DIM), jnp.float32),
            pltpu.VMEM((ROWS, DIM), jnp.float32),
            pltpu.VMEM((BPW, L), jnp.float32),
            pltpu.VMEM((BPW, NPAD), jnp.float32),
            pltpu.VMEM((BPW, NPAD), jnp.float32),
            pltpu.SemaphoreType.DMA,
            pltpu.SemaphoreType.DMA,
        ],
        compiler_params=pltpu.CompilerParams(use_tc_tiling_on_sc=False),
    )
    return f(table, idx_flat)


def _tc_body(scal_ref, nd_ref, nn_ref, out_ref):
    eps = 1e-8
    pos_dot = scal_ref[:, 0:1]
    m2 = scal_ref[:, 1:2]
    tt = scal_ref[:, 2:3]
    na = jnp.maximum(jnp.sqrt(tt), eps)
    nb = jnp.maximum(jnp.sqrt(m2) * (1.0 / HIST), eps)
    pos_sim = (pos_dot * (1.0 / HIST)) / (na * nb)

    nd = nd_ref[:]
    nn = nn_ref[:]
    nbn = jnp.maximum(jnp.sqrt(nn), eps)
    neg_sim = nd / (na * nbn)
    col = lax.broadcasted_iota(jnp.int32, (B, NPAD), 1)
    neg_sim = jnp.where(col < N_NEG, neg_sim, -1e30)

    logits = jnp.concatenate([pos_sim, neg_sim], axis=1)
    mx = jnp.max(logits, axis=1, keepdims=True)
    lse = mx + jnp.log(jnp.sum(jnp.exp(logits - mx), axis=1, keepdims=True))
    logp0 = pos_sim - lse
    out_ref[...] = jnp.reshape(-jnp.sum(logp0) * (1.0 / B), (1, 1))


def _tc_loss(scal, nd, nn):
    return pl.pallas_call(
        _tc_body,
        out_shape=jax.ShapeDtypeStruct((1, 1), jnp.float32),
    )(scal, nd, nn)


def _tc_idx_body(tr_ref, ng_ref, tg_ref, out_ref):
    tg = tg_ref[...]
    out_ref[...] = jnp.concatenate(
        [tr_ref[...], ng_ref[...], tg, tg, tg, tg], axis=1)


def _tc_build_idx(train_idx, neg_idx, target_idx):
    return pl.pallas_call(
        _tc_idx_body,
        out_shape=jax.ShapeDtypeStruct((B, ROWS), jnp.int32),
    )(train_idx, neg_idx, target_idx)


def kernel(table, train_idx, neg_idx, target_idx):
    idx2d = _tc_build_idx(train_idx.astype(jnp.int32),
                          neg_idx.astype(jnp.int32),
                          target_idx.astype(jnp.int32)[:, None])
    scal, nd, nn = _sc_gather_reduce(table, idx2d)
    loss = _tc_loss(scal, nd, nn)
    return jnp.reshape(loss, ())


# speedup vs baseline: 1.5112x; 1.0008x over previous
import jax
import jax.numpy as jnp
from jax import lax
from jax.experimental import pallas as pl
from jax.experimental.pallas import tpu as pltpu
from jax.experimental.pallas import tpu_sc as plsc

M_ITEM = 1000000
DIM = 64
B = 1024
HIST = 200
N_NEG = 100

L = 16
NPAD = 104
ROWS = 304
TGT_ROW = HIST + N_NEG
NEG_STARTS = (0, 16, 32, 48, 64, 80, 88)

NC = 2
NS = 16
NW = NC * NS
BPW = B // NW


def _shuffle(v, idx16):
    return lax.gather(
        v, idx16[:, None],
        lax.GatherDimensionNumbers(offset_dims=(), collapsed_slice_dims=(0,),
                                   start_index_map=(0,)),
        (1,), mode=lax.GatherScatterMode.PROMISE_IN_BOUNDS)


def _lanesum(v, lanes):
    for k in (8, 4, 2, 1):
        v = v + _shuffle(v, lanes ^ k)
    return v


def _merge_tree(vs, lanes):
    for k in (1, 2, 4, 8):
        nxt = []
        for i in range(0, len(vs), 2):
            a, b = vs[i], vs[i + 1]
            asum = a + _shuffle(a, lanes ^ k)
            bsum = b + _shuffle(b, lanes ^ k)
            nxt.append(jnp.where((lanes & k) == 0, asum, bsum))
        vs = nxt
    return vs[0]


def _sc_body(table, idx, scal_out, nd_out, nn_out,
             idx_v, rows_a, rows_b, scal_v, nd_v, nn_v, sem_a, sem_b):
    wid = lax.axis_index("s") * NC + lax.axis_index("c")
    base = wid * BPW

    pltpu.sync_copy(idx.at[pl.ds(base, BPW)], idx_v)

    lanes = jnp.arange(L, dtype=jnp.int32)
    zero = jnp.zeros((L,), jnp.float32)

    def _copies(bi, rows_v, sem):
        return (
            (idx_v.at[bi, pl.ds(0, 128)], rows_v.at[pl.ds(0, 128)], sem),
            (idx_v.at[bi, pl.ds(128, 128)], rows_v.at[pl.ds(128, 128)], sem),
            (idx_v.at[bi, pl.ds(256, 48)], rows_v.at[pl.ds(256, 48)], sem),
        )

    def fire(bi, rows_v, sem):
        for i_ref, d_ref, s in _copies(bi, rows_v, sem):
            pltpu.async_copy(table.at[i_ref], d_ref, s)

    def drain(bi, rows_v, sem):
        for i_ref, d_ref, s in _copies(bi, rows_v, sem):
            pltpu.make_async_copy(table.at[i_ref], d_ref, s).wait()

    def compute(bi, rows_v):
        t0 = rows_v[TGT_ROW, pl.ds(0, L)]
        t1 = rows_v[TGT_ROW, pl.ds(L, L)]
        t2 = rows_v[TGT_ROW, pl.ds(2 * L, L)]
        t3 = rows_v[TGT_ROW, pl.ds(3 * L, L)]

        def hist_body(h, acc):
            a0, a1, a2, a3 = acc
            for u in range(4):
                r = h * 4 + u
                a0 = a0 + rows_v[r, pl.ds(0, L)]
                a1 = a1 + rows_v[r, pl.ds(L, L)]
                a2 = a2 + rows_v[r, pl.ds(2 * L, L)]
                a3 = a3 + rows_v[r, pl.ds(3 * L, L)]
            return a0, a1, a2, a3

        a0, a1, a2, a3 = lax.fori_loop(0, HIST // 4, hist_body,
                                       (zero, zero, zero, zero))

        pos = _lanesum(a0 * t0 + a1 * t1 + a2 * t2 + a3 * t3, lanes)
        m2 = _lanesum(a0 * a0 + a1 * a1 + a2 * a2 + a3 * a3, lanes)
        tt = _lanesum(t0 * t0 + t1 * t1 + t2 * t2 + t3 * t3, lanes)
        sv = jnp.where(lanes == 0, pos,
                       jnp.where(lanes == 1, m2,
                                 jnp.where(lanes == 2, tt, zero)))
        scal_v[bi, pl.ds(0, L)] = sv

        for g0 in NEG_STARTS:
            dps = []
            nps = []
            for j in range(L):
                r = HIST + g0 + j
                n0 = rows_v[r, pl.ds(0, L)]
                n1 = rows_v[r, pl.ds(L, L)]
                n2 = rows_v[r, pl.ds(2 * L, L)]
                n3 = rows_v[r, pl.ds(3 * L, L)]
                dps.append(n0 * t0 + n1 * t1 + n2 * t2 + n3 * t3)
                nps.append(n0 * n0 + n1 * n1 + n2 * n2 + n3 * n3)
            nd_v[bi, pl.ds(g0, L)] = _merge_tree(dps, lanes)
            nn_v[bi, pl.ds(g0, L)] = _merge_tree(nps, lanes)

    fire(0, rows_a, sem_a)

    def pair_body(p, carry):
        b0 = 2 * p
        fire(b0 + 1, rows_b, sem_b)
        drain(b0, rows_a, sem_a)
        compute(b0, rows_a)
        fire(lax.rem(b0 + 2, BPW), rows_a, sem_a)
        drain(b0 + 1, rows_b, sem_b)
        compute(b0 + 1, rows_b)
        return carry

    lax.fori_loop(0, BPW // 2, pair_body, 0)
    drain(0, rows_a, sem_a)

    pltpu.sync_copy(scal_v, scal_out.at[pl.ds(base, BPW)])
    pltpu.sync_copy(nd_v, nd_out.at[pl.ds(base, BPW)])
    pltpu.sync_copy(nn_v, nn_out.at[pl.ds(base, BPW)])


def _sc_gather_reduce(table, idx_flat):
    mesh = plsc.VectorSubcoreMesh(core_axis_name="c", subcore_axis_name="s")
    f = pl.kernel(
        _sc_body,
        mesh=mesh,
        out_type=(
            jax.ShapeDtypeStruct((B, L), jnp.float32),
            jax.ShapeDtypeStruct((B, NPAD), jnp.float32),
            jax.ShapeDtypeStruct((B, NPAD), jnp.float32),
        ),
        scratch_types=[
            pltpu.VMEM((BPW, ROWS), jnp.int32),
            pltpu.VMEM((ROWS, DIM), jnp.float32),
            pltpu.VMEM((ROWS, DIM), jnp.float32),
            pltpu.VMEM((BPW, L), jnp.float32),
            pltpu.VMEM((BPW, NPAD), jnp.float32),
            pltpu.VMEM((BPW, NPAD), jnp.float32),
            pltpu.SemaphoreType.DMA,
            pltpu.SemaphoreType.DMA,
        ],
        compiler_params=pltpu.CompilerParams(use_tc_tiling_on_sc=False),
    )
    return f(table, idx_flat)


def _tc_body(scal_ref, nd_ref, nn_ref, out_ref):
    eps = 1e-8
    pos_dot = scal_ref[:, 0:1]
    m2 = scal_ref[:, 1:2]
    tt = scal_ref[:, 2:3]
    na = jnp.maximum(jnp.sqrt(tt), eps)
    nb = jnp.maximum(jnp.sqrt(m2) * (1.0 / HIST), eps)
    pos_sim = (pos_dot * (1.0 / HIST)) / (na * nb)

    nd = nd_ref[:]
    nn = nn_ref[:]
    nbn = jnp.maximum(jnp.sqrt(nn), eps)
    neg_sim = nd / (na * nbn)
    col = lax.broadcasted_iota(jnp.int32, (B, NPAD), 1)
    neg_sim = jnp.where(col < N_NEG, neg_sim, -1e30)

    logits = jnp.concatenate([pos_sim, neg_sim], axis=1)
    mx = jnp.max(logits, axis=1, keepdims=True)
    lse = mx + jnp.log(jnp.sum(jnp.exp(logits - mx), axis=1, keepdims=True))
    logp0 = pos_sim - lse
    out_ref[...] = jnp.reshape(-jnp.sum(logp0) * (1.0 / B), (1, 1))


def _tc_loss(scal, nd, nn):
    return pl.pallas_call(
        _tc_body,
        out_shape=jax.ShapeDtypeStruct((1, 1), jnp.float32),
    )(scal, nd, nn)


def _tc_idx_body(tr_ref, ng_ref, tg_ref, out_ref):
    tg = tg_ref[...]
    out_ref[...] = jnp.concatenate(
        [tr_ref[...], ng_ref[...], tg, tg, tg, tg], axis=1)


def _tc_build_idx(train_idx, neg_idx, target_idx):
    return pl.pallas_call(
        _tc_idx_body,
        out_shape=jax.ShapeDtypeStruct((B, ROWS), jnp.int32),
    )(train_idx, neg_idx, target_idx)


def kernel(table, train_idx, neg_idx, target_idx):
    idx2d = _tc_build_idx(train_idx.astype(jnp.int32),
                          neg_idx.astype(jnp.int32),
                          target_idx.astype(jnp.int32)[:, None])
    scal, nd, nn = _sc_gather_reduce(table, idx2d)
    loss = _tc_loss(scal, nd, nn)
    return jnp.reshape(loss, ())
